# baseline (speedup 1.0000x reference)
import functools

import jax
import jax.numpy as jnp
from jax import lax
from jax.experimental import pallas as pl
from jax.experimental.pallas import tpu as pltpu
from jax.experimental.pallas import tpu_sc as plsc

_L = 16
_NC = 2
_NS = 16
_NW = _NC * _NS
_CHUNK = 128


@functools.lru_cache(maxsize=None)
def _build(n):
    h = n // 2
    per_w = -(-n // (_NW * _CHUNK)) * _CHUNK
    half_w = per_w // 2
    n_chunks = per_w // _CHUNK
    assert n >= per_w and (h - half_w) % 8 == 0
    mesh = plsc.VectorSubcoreMesh(core_axis_name="c", subcore_axis_name="s")

    @functools.partial(
        pl.kernel,
        mesh=mesh,
        out_type=(jax.ShapeDtypeStruct((h,), jnp.float32),
                  jax.ShapeDtypeStruct((h,), jnp.float32)),
        scratch_types=[
            pltpu.VMEM((n_chunks, _CHUNK), jnp.int32),
            pltpu.VMEM((per_w,), jnp.float32),
            pltpu.VMEM((half_w,), jnp.float32),
            pltpu.VMEM((half_w,), jnp.float32),
            pltpu.SemaphoreType.DMA,
            pltpu.SemaphoreType.DMA,
        ],
        compiler_params=pltpu.CompilerParams(needs_layout_passes=False),
    )
    def gather_kernel(xflat, lin_hbm, outb_hbm, outd_hbm,
                      lin_v, vals_v, b_v, d_v, sem, sem_idx):
        wid = lax.axis_index("s") * _NC + lax.axis_index("c")
        base = pl.multiple_of(jnp.minimum(wid * half_w, h - half_w), 8)
        loads = [
            pltpu.async_copy(
                lin_hbm.at[pl.ds(2 * base + j * _CHUNK, _CHUNK)],
                lin_v.at[j], sem_idx)
            for j in range(n_chunks)
        ]
        for ld in loads:
            ld.wait()
        copies = [
            pltpu.async_copy(xflat.at[lin_v.at[j]],
                             vals_v.at[pl.ds(j * _CHUNK, _CHUNK)], sem)
            for j in range(n_chunks)
        ]
        for cp in copies:
            cp.wait()
        lane2 = lax.iota(jnp.int32, _L) * 2
        for g in range(half_w // _L):
            e = lane2 + 2 * _L * g
            b_v[pl.ds(g * _L, _L)] = plsc.load_gather(vals_v, [e])
            d_v[pl.ds(g * _L, _L)] = plsc.load_gather(vals_v, [e + 1])
        pltpu.sync_copy(b_v, outb_hbm.at[pl.ds(base, half_w)])
        pltpu.sync_copy(d_v, outd_hbm.at[pl.ds(base, half_w)])

    return gather_kernel


def kernel(X, indices):
    n = indices.shape[0]
    h = n // 2
    nr, nc = X.shape
    r, c = indices[:, 0], indices[:, 1]
    if nr % 8 == 0 and nc % 128 == 0:
        lin = (((r >> 3) * (nc // 128) + (c >> 7)) << 10) + \
              ((r & 7) << 7) + (c & 127)
        xflat = X.reshape(nr // 8, 8, nc // 128, 128)
        xflat = xflat.transpose(0, 2, 1, 3).reshape(-1)
    else:
        lin = r * nc + c
        xflat = X.reshape(-1)
    births, deaths = _build(n)(xflat, lin)
    return jnp.stack([births, deaths], axis=1)


# speedup vs baseline: 1.0323x; 1.0323x over previous
import functools

import jax
import jax.numpy as jnp
from jax import lax
from jax.experimental import pallas as pl
from jax.experimental.pallas import tpu as pltpu
from jax.experimental.pallas import tpu_sc as plsc

_L = 16
_NC = 2
_NS = 16
_NW = _NC * _NS
_CHUNK = 128


@functools.lru_cache(maxsize=None)
def _build(n):
    h = n // 2
    per_w = -(-n // (_NW * _CHUNK)) * _CHUNK
    half_w = per_w // 2
    n_chunks = per_w // _CHUNK
    assert n >= per_w and (h - half_w) % 8 == 0
    mesh = plsc.VectorSubcoreMesh(core_axis_name="c", subcore_axis_name="s")

    @functools.partial(
        pl.kernel,
        mesh=mesh,
        out_type=(jax.ShapeDtypeStruct((h,), jnp.float32),
                  jax.ShapeDtypeStruct((h,), jnp.float32)),
        scratch_types=[
            pltpu.VMEM((n_chunks, _CHUNK), jnp.int32),
            pltpu.VMEM((per_w,), jnp.float32),
            pltpu.VMEM((half_w,), jnp.float32),
            pltpu.VMEM((half_w,), jnp.float32),
        ] + [pltpu.SemaphoreType.DMA] * (2 * n_chunks + 1),
        compiler_params=pltpu.CompilerParams(needs_layout_passes=False),
    )
    def gather_kernel(xflat, lin_hbm, outb_hbm, outd_hbm,
                      lin_v, vals_v, b_v, d_v, *sems):
        wid = lax.axis_index("s") * _NC + lax.axis_index("c")
        base = pl.multiple_of(jnp.minimum(wid * half_w, h - half_w), 8)
        lane2 = lax.iota(jnp.int32, _L) * 2
        gpc = _CHUNK // (2 * _L)

        def deinterleave(j):
            for i in range(gpc):
                g = j * gpc + i
                e = lane2 + 2 * _L * g
                b_v[pl.ds(g * _L, _L)] = plsc.load_gather(vals_v, [e])
                d_v[pl.ds(g * _L, _L)] = plsc.load_gather(vals_v, [e + 1])

        loads = [
            pltpu.async_copy(
                lin_hbm.at[pl.ds(2 * base + j * _CHUNK, _CHUNK)],
                lin_v.at[j], sems[j])
            for j in range(n_chunks)
        ]
        gathers = []
        for j in range(n_chunks):
            loads[j].wait()
            gathers.append(
                pltpu.async_copy(xflat.at[lin_v.at[j]],
                                 vals_v.at[pl.ds(j * _CHUNK, _CHUNK)],
                                 sems[n_chunks + j]))
        for j in range(n_chunks):
            gathers[j].wait()
            deinterleave(j)
        ob = pltpu.async_copy(b_v, outb_hbm.at[pl.ds(base, half_w)],
                              sems[2 * n_chunks])
        od = pltpu.async_copy(d_v, outd_hbm.at[pl.ds(base, half_w)],
                              sems[2 * n_chunks])
        ob.wait()
        od.wait()

    return gather_kernel


def kernel(X, indices):
    n = indices.shape[0]
    h = n // 2
    nr, nc = X.shape
    r, c = indices[:, 0], indices[:, 1]
    if nr % 8 == 0 and nc % 128 == 0:
        lin = (((r >> 3) * (nc // 128) + (c >> 7)) << 10) + \
              ((r & 7) << 7) + (c & 127)
        xflat = X.reshape(nr // 8, 8, nc // 128, 128)
        xflat = xflat.transpose(0, 2, 1, 3).reshape(-1)
    else:
        lin = r * nc + c
        xflat = X.reshape(-1)
    births, deaths = _build(n)(xflat, lin)
    return jnp.stack([births, deaths], axis=1)
